# Initial kernel scaffold; baseline (speedup 1.0000x reference)
import jax
import jax.numpy as jnp
from jax.experimental import pallas as pl


def kernel(x, edge_index, edge_attr, batch, params):
    raise NotImplementedError("write your pallas kernel here")


# speedup vs baseline: 1.0014x; 1.0014x over previous
import functools

import jax
import jax.numpy as jnp
from jax import lax
from jax.experimental import pallas as pl
from jax.experimental.pallas import tpu as pltpu

H = 128
ED = 16
B = 512
NP = 10240
N = 10000
E = 320000

_f32 = jnp.float32


def _mm_body(x_ref, w_ref, b_ref, o_ref, *, act):
    y = jnp.dot(x_ref[...], w_ref[...], preferred_element_type=_f32)
    y = y + b_ref[...]
    if act == 'relu':
        y = jnp.maximum(y, 0.0)
    o_ref[...] = y


def _mm(x, wT, b, act='none', rb=1024):
    M, K = x.shape
    Nout = wT.shape[1]
    grid = M // rb
    return pl.pallas_call(
        functools.partial(_mm_body, act=act),
        grid=(grid,),
        in_specs=[
            pl.BlockSpec((rb, K), lambda i: (i, 0)),
            pl.BlockSpec((K, Nout), lambda i: (0, 0)),
            pl.BlockSpec((1, Nout), lambda i: (0, 0)),
        ],
        out_specs=pl.BlockSpec((rb, Nout), lambda i: (i, 0)),
        out_shape=jax.ShapeDtypeStruct((M, Nout), _f32),
    )(x, wT, b.reshape(1, Nout))


def _rowdot_body(m_ref, v_ref, o_ref):
    blk = m_ref[0]
    o_ref[0] = lax.dot_general(
        v_ref[...], blk, (((1,), (1,)), ((), ())),
        preferred_element_type=_f32)


def _rowdots(mat, vecs):
    M = mat.shape[0]
    V = vecs.shape[0]
    G = M // 128
    mat3 = mat.reshape(G, 128, 128)
    out = pl.pallas_call(
        _rowdot_body,
        grid=(G,),
        in_specs=[
            pl.BlockSpec((1, 128, 128), lambda i: (i, 0, 0)),
            pl.BlockSpec((V, 128), lambda i: (0, 0)),
        ],
        out_specs=pl.BlockSpec((1, V, 128), lambda i: (i, 0, 0)),
        out_shape=jax.ShapeDtypeStruct((G, V, 128), _f32),
    )(mat3, vecs)
    return out.transpose(1, 0, 2).reshape(V, M)


def _eaq_body(ea_ref, m_ref, ones_ref, o_ref):
    o_ref[...] = jnp.dot(ea_ref[...], m_ref[...],
                         preferred_element_type=_f32) + ones_ref[...]


def _build_eaq(ea2, Mmat, onesrow, rb=2000):
    M = ea2.shape[0]
    return pl.pallas_call(
        _eaq_body,
        grid=(M // rb,),
        in_specs=[
            pl.BlockSpec((rb, 128), lambda i: (i, 0)),
            pl.BlockSpec((128, 256), lambda i: (0, 0)),
            pl.BlockSpec((1, 256), lambda i: (0, 0)),
        ],
        out_specs=pl.BlockSpec((rb, 256), lambda i: (i, 0)),
        out_shape=jax.ShapeDtypeStruct((M, 256), _f32),
    )(ea2, Mmat, onesrow)


def _sigmoid(x):
    return 1.0 / (1.0 + jnp.exp(-x))


def _gru_from(gi, gh, h):
    ir, iz, inn = gi[:, :H], gi[:, H:2 * H], gi[:, 2 * H:]
    hr, hz, hn = gh[:, :H], gh[:, H:2 * H], gh[:, 2 * H:]
    r = _sigmoid(ir + hr)
    z = _sigmoid(iz + hz)
    nn_ = jnp.tanh(inn + r * hn)
    return (1.0 - z) * nn_ + z * h


def _gate_update_body(p0_ref, p1_ref, q0_ref, q1_ref, x0_ref,
                      wle_ref, bias_ref, wih_ref, whh_ref, bih_ref, bhh_ref,
                      o_ref):
    qd = q0_ref[...] + q1_ref[...]
    P = p0_ref[...] + p1_ref[...]
    den = qd[:, 16:17] + 1e-16
    h = (P + jnp.dot(qd[:, :16], wle_ref[...],
                     preferred_element_type=_f32)) / den + bias_ref[...]
    h = jnp.where(h > 0, h, jnp.exp(jnp.minimum(h, 0.0)) - 1.0)
    x0 = x0_ref[...]
    gi = jnp.dot(h, wih_ref[...], preferred_element_type=_f32) + bih_ref[...]
    gh = jnp.dot(x0, whh_ref[...], preferred_element_type=_f32) + bhh_ref[...]
    o_ref[...] = _gru_from(gi, gh, x0)


def _gate_update(P0, P1, QD0, QD1, x0, wleT, bias, wihT, whhT, bih, bhh,
                 rb=1024):
    M = x0.shape[0]
    return pl.pallas_call(
        _gate_update_body,
        grid=(M // rb,),
        in_specs=[
            pl.BlockSpec((rb, 128), lambda i: (i, 0)),
            pl.BlockSpec((rb, 128), lambda i: (i, 0)),
            pl.BlockSpec((rb, 32), lambda i: (i, 0)),
            pl.BlockSpec((rb, 32), lambda i: (i, 0)),
            pl.BlockSpec((rb, 128), lambda i: (i, 0)),
            pl.BlockSpec((16, 128), lambda i: (0, 0)),
            pl.BlockSpec((1, 128), lambda i: (0, 0)),
            pl.BlockSpec((128, 384), lambda i: (0, 0)),
            pl.BlockSpec((128, 384), lambda i: (0, 0)),
            pl.BlockSpec((1, 384), lambda i: (0, 0)),
            pl.BlockSpec((1, 384), lambda i: (0, 0)),
        ],
        out_specs=pl.BlockSpec((rb, 128), lambda i: (i, 0)),
        out_shape=jax.ShapeDtypeStruct((M, 128), _f32),
    )(P0, P1, QD0, QD1, x0, wleT, bias.reshape(1, 128),
      wihT, whhT, bih.reshape(1, 384), bhh.reshape(1, 384))


def _agg_update_body(p0_ref, p1_ref, d0_ref, d1_ref, xp_ref,
                     bias_ref, wih_ref, whh_ref, bih_ref, bhh_ref, o_ref):
    den = d0_ref[:, 0:1] + d1_ref[:, 0:1] + 1e-16
    h = (p0_ref[...] + p1_ref[...]) / den + bias_ref[...]
    h = jnp.where(h > 0, h, jnp.exp(jnp.minimum(h, 0.0)) - 1.0)
    xp = xp_ref[...]
    gi = jnp.dot(h, wih_ref[...], preferred_element_type=_f32) + bih_ref[...]
    gh = jnp.dot(xp, whh_ref[...], preferred_element_type=_f32) + bhh_ref[...]
    o_ref[...] = _gru_from(gi, gh, xp)


def _agg_update(P0, P1, D0, D1, xprev, bias, wihT, whhT, bih, bhh, rb=1024):
    M = xprev.shape[0]
    return pl.pallas_call(
        _agg_update_body,
        grid=(M // rb,),
        in_specs=[
            pl.BlockSpec((rb, 128), lambda i: (i, 0)),
            pl.BlockSpec((rb, 128), lambda i: (i, 0)),
            pl.BlockSpec((rb, 16), lambda i: (i, 0)),
            pl.BlockSpec((rb, 16), lambda i: (i, 0)),
            pl.BlockSpec((rb, 128), lambda i: (i, 0)),
            pl.BlockSpec((1, 128), lambda i: (0, 0)),
            pl.BlockSpec((128, 384), lambda i: (0, 0)),
            pl.BlockSpec((128, 384), lambda i: (0, 0)),
            pl.BlockSpec((1, 384), lambda i: (0, 0)),
            pl.BlockSpec((1, 384), lambda i: (0, 0)),
        ],
        out_specs=pl.BlockSpec((rb, 128), lambda i: (i, 0)),
        out_shape=jax.ShapeDtypeStruct((M, 128), _f32),
    )(P0, P1, D0, D1, xprev, bias.reshape(1, 128),
      wihT, whhT, bih.reshape(1, 384), bhh.reshape(1, 384))


def _edge_aggregate_xla(s, r, q, src, dst, t, n_dst, extra=None):
    z = s[src] + (q if q is not None else 0.0) + r[dst]
    zl = jnp.where(z > 0, z, 0.2 * z)
    c = jnp.maximum(r, 0.0)
    al = jnp.exp(zl - c[dst])
    den = jax.ops.segment_sum(al, dst, num_segments=n_dst)
    P = jax.ops.segment_sum(t[src] * al[:, None], dst, num_segments=n_dst)
    Q = None
    if extra is not None:
        Q = jax.ops.segment_sum(extra * al[:, None], dst, num_segments=n_dst)
    return P, den, Q


def kernel(x, edge_index, edge_attr, batch, params):
    p = params
    src, dst = edge_index[0], edge_index[1]

    xp = jnp.pad(x, ((0, NP - N), (0, 0)))
    x0 = _mm(xp, p['lin1_W'].T, p['lin1_b'], act='relu')

    Wl = p['gate_lin_l_W']
    WlxT, Wle = Wl[:, :H].T, Wl[:, H:]
    t = _mm(x0, WlxT, jnp.zeros((H,), _f32))
    sr = _rowdots(t, p['gate_att_l'].reshape(1, 128))
    s = sr[0]
    vr = p['gate_lin_r_W'].T @ p['gate_att_r']
    r = _rowdots(x0, vr.reshape(1, 128))[0]

    wq = Wle.T @ p['gate_att_l']
    Mmat = jnp.zeros((128, 256), _f32)
    for j in range(8):
        Mmat = Mmat.at[16 * j:16 * j + 16, 32 * j:32 * j + 16].set(
            jnp.eye(16, dtype=_f32))
        Mmat = Mmat.at[16 * j:16 * j + 16, 32 * j + 17].set(wq)
    onesrow = jnp.zeros((1, 256), _f32)
    for j in range(8):
        onesrow = onesrow.at[0, 32 * j + 16].set(1.0)
    eaq = _build_eaq(edge_attr.reshape(E // 8, 128), Mmat, onesrow)
    eaq = eaq.reshape(E, 32)
    q = eaq[:, 17]

    Pg, deng, Qg = _edge_aggregate_xla(s, r, q, src, dst, t, NP,
                                       extra=edge_attr)
    QDg = jnp.concatenate(
        [Qg, deng[:, None], jnp.zeros((NP, 15), _f32)], axis=1)
    zeros128 = jnp.zeros((NP, 128), _f32)
    zeros32 = jnp.zeros((NP, 32), _f32)
    x1 = _gate_update(Pg, zeros128, QDg, zeros32, x0,
                      Wle.T.reshape(16, 128), p['gate_bias'],
                      p['agru0_Wih'].T, p['agru0_Whh'].T,
                      p['agru0_bih'], p['agru0_bhh'])

    hh = _mm(x1, p['aconv1_W'].T, jnp.zeros((H,), _f32))
    sr2 = _rowdots(hh, jnp.stack([p['aconv1_att_src'], p['aconv1_att_dst']]))
    s2, r2 = sr2[0], sr2[1]
    Pa, dena, _ = _edge_aggregate_xla(s2, r2, None, src, dst, hh, NP)
    Da = jnp.concatenate([dena[:, None], jnp.zeros((NP, 15), _f32)], axis=1)
    zeros16 = jnp.zeros((NP, 16), _f32)
    x2 = _agg_update(Pa, zeros128, Da, zeros16, x1,
                     p['aconv1_bias'],
                     p['agru1_Wih'].T, p['agru1_Whh'].T,
                     p['agru1_bih'], p['agru1_bhh'])

    BP = 512
    batch_p = jnp.pad(batch, (0, NP - N), constant_values=B).astype(jnp.int32)
    xs = [x0, x1, x2]
    zeroN = jnp.zeros((NP,), _f32)
    zeroB = jnp.zeros((BP,), _f32)
    P0m, d0m, _ = _edge_aggregate_xla(
        zeroN, zeroB, None, jnp.arange(NP), batch_p, x2, BP + 16)
    out = P0m[:BP]
    for i in range(3):
        W = p['mconv%d_W' % i]
        hs = _mm(xs[i], W.T, jnp.zeros((H,), _f32))
        sm = _rowdots(hs, p['mconv%d_att_src' % i].reshape(1, 128))[0]
        hd = _mm(out, W.T, jnp.zeros((H,), _f32), rb=512)
        rm = _rowdots(hd, p['mconv%d_att_dst' % i].reshape(1, 128))[0]
        rm_p = jnp.pad(rm, (0, 16))
        Pm, dm, _ = _edge_aggregate_xla(sm, rm_p, None, jnp.arange(NP),
                                        batch_p, hs, BP + 16)
        Dm = jnp.concatenate(
            [dm[:BP, None], jnp.zeros((BP, 15), _f32)], axis=1)
        zeros16b = jnp.zeros((BP, 16), _f32)
        zeros128b = jnp.zeros((BP, 128), _f32)
        out = _agg_update(Pm[:BP], zeros128b, Dm, zeros16b, out,
                          p['mconv%d_bias' % i],
                          p['mgru%d_Wih' % i].T, p['mgru%d_Whh' % i].T,
                          p['mgru%d_bih' % i], p['mgru%d_bhh' % i], rb=512)

    return _mm(out, p['lin2_W'].T, p['lin2_b'], rb=512)


# speedup vs baseline: 6.7238x; 6.7147x over previous
import functools

import jax
import jax.numpy as jnp
from jax import lax
from jax.experimental import pallas as pl
from jax.experimental.pallas import tpu as pltpu
from jax.experimental.pallas import tpu_sc as plsc

H = 128
ED = 16
B = 512
NP = 10240
N = 10000
E = 320000

_f32 = jnp.float32


def _mm_body(x_ref, w_ref, b_ref, o_ref, *, act):
    y = jnp.dot(x_ref[...], w_ref[...], preferred_element_type=_f32)
    y = y + b_ref[...]
    if act == 'relu':
        y = jnp.maximum(y, 0.0)
    o_ref[...] = y


def _mm(x, wT, b, act='none', rb=1024):
    M, K = x.shape
    Nout = wT.shape[1]
    grid = M // rb
    return pl.pallas_call(
        functools.partial(_mm_body, act=act),
        grid=(grid,),
        in_specs=[
            pl.BlockSpec((rb, K), lambda i: (i, 0)),
            pl.BlockSpec((K, Nout), lambda i: (0, 0)),
            pl.BlockSpec((1, Nout), lambda i: (0, 0)),
        ],
        out_specs=pl.BlockSpec((rb, Nout), lambda i: (i, 0)),
        out_shape=jax.ShapeDtypeStruct((M, Nout), _f32),
    )(x, wT, b.reshape(1, Nout))


def _rowdot_body(m_ref, v_ref, o_ref):
    blk = m_ref[0]
    o_ref[0] = lax.dot_general(
        v_ref[...], blk, (((1,), (1,)), ((), ())),
        preferred_element_type=_f32)


def _rowdots(mat, vecs):
    M = mat.shape[0]
    V = vecs.shape[0]
    G = M // 128
    mat3 = mat.reshape(G, 128, 128)
    out = pl.pallas_call(
        _rowdot_body,
        grid=(G,),
        in_specs=[
            pl.BlockSpec((1, 128, 128), lambda i: (i, 0, 0)),
            pl.BlockSpec((V, 128), lambda i: (0, 0)),
        ],
        out_specs=pl.BlockSpec((1, V, 128), lambda i: (i, 0, 0)),
        out_shape=jax.ShapeDtypeStruct((G, V, 128), _f32),
    )(mat3, vecs)
    return out.transpose(1, 0, 2).reshape(V, M)


def _eaq_body(ea_ref, m_ref, ones_ref, o_ref):
    o_ref[...] = jnp.dot(ea_ref[...], m_ref[...],
                         preferred_element_type=_f32) + ones_ref[...]


def _build_eaq(ea2, Mmat, onesrow, rb=2000):
    M = ea2.shape[0]
    return pl.pallas_call(
        _eaq_body,
        grid=(M // rb,),
        in_specs=[
            pl.BlockSpec((rb, 128), lambda i: (i, 0)),
            pl.BlockSpec((128, 256), lambda i: (0, 0)),
            pl.BlockSpec((1, 256), lambda i: (0, 0)),
        ],
        out_specs=pl.BlockSpec((rb, 256), lambda i: (i, 0)),
        out_shape=jax.ShapeDtypeStruct((M, 256), _f32),
    )(ea2, Mmat, onesrow)


def _sigmoid(x):
    return 1.0 / (1.0 + jnp.exp(-x))


def _gru_from(gi, gh, h):
    ir, iz, inn = gi[:, :H], gi[:, H:2 * H], gi[:, 2 * H:]
    hr, hz, hn = gh[:, :H], gh[:, H:2 * H], gh[:, 2 * H:]
    r = _sigmoid(ir + hr)
    z = _sigmoid(iz + hz)
    nn_ = jnp.tanh(inn + r * hn)
    return (1.0 - z) * nn_ + z * h


def _gate_update_body(p0_ref, p1_ref, qd_ref, x0_ref,
                      wle_ref, bias_ref, wih_ref, whh_ref, bih_ref, bhh_ref,
                      o_ref):
    qd = qd_ref[...]
    P = jnp.concatenate([p0_ref[...], p1_ref[...]], axis=1)
    den = qd[:, 16:17] + 1e-16
    h = (P + jnp.dot(qd[:, :16], wle_ref[...],
                     preferred_element_type=_f32)) / den + bias_ref[...]
    h = jnp.where(h > 0, h, jnp.exp(jnp.minimum(h, 0.0)) - 1.0)
    x0 = x0_ref[...]
    gi = jnp.dot(h, wih_ref[...], preferred_element_type=_f32) + bih_ref[...]
    gh = jnp.dot(x0, whh_ref[...], preferred_element_type=_f32) + bhh_ref[...]
    o_ref[...] = _gru_from(gi, gh, x0)


def _gate_update(P0, P1, QD, x0, wleT, bias, wihT, whhT, bih, bhh,
                 rb=1000):
    M = x0.shape[0]
    return pl.pallas_call(
        _gate_update_body,
        grid=(M // rb,),
        in_specs=[
            pl.BlockSpec((rb, 64), lambda i: (i, 0)),
            pl.BlockSpec((rb, 64), lambda i: (i, 0)),
            pl.BlockSpec((rb, 32), lambda i: (i, 0)),
            pl.BlockSpec((rb, 128), lambda i: (i, 0)),
            pl.BlockSpec((16, 128), lambda i: (0, 0)),
            pl.BlockSpec((1, 128), lambda i: (0, 0)),
            pl.BlockSpec((128, 384), lambda i: (0, 0)),
            pl.BlockSpec((128, 384), lambda i: (0, 0)),
            pl.BlockSpec((1, 384), lambda i: (0, 0)),
            pl.BlockSpec((1, 384), lambda i: (0, 0)),
        ],
        out_specs=pl.BlockSpec((rb, 128), lambda i: (i, 0)),
        out_shape=jax.ShapeDtypeStruct((M, 128), _f32),
    )(P0, P1, QD, x0, wleT, bias.reshape(1, 128),
      wihT, whhT, bih.reshape(1, 384), bhh.reshape(1, 384))


def _agg_update_body(p0_ref, p1_ref, d_ref, xp_ref,
                     bias_ref, wih_ref, whh_ref, bih_ref, bhh_ref, o_ref):
    den = d_ref[:, 0:1] + 1e-16
    P = jnp.concatenate([p0_ref[...], p1_ref[...]], axis=1)
    h = P / den + bias_ref[...]
    h = jnp.where(h > 0, h, jnp.exp(jnp.minimum(h, 0.0)) - 1.0)
    xp = xp_ref[...]
    gi = jnp.dot(h, wih_ref[...], preferred_element_type=_f32) + bih_ref[...]
    gh = jnp.dot(xp, whh_ref[...], preferred_element_type=_f32) + bhh_ref[...]
    o_ref[...] = _gru_from(gi, gh, xp)


def _agg_update(P0, P1, D, xprev, bias, wihT, whhT, bih, bhh, rb=1000):
    M = xprev.shape[0]
    return pl.pallas_call(
        _agg_update_body,
        grid=(M // rb,),
        in_specs=[
            pl.BlockSpec((rb, 64), lambda i: (i, 0)),
            pl.BlockSpec((rb, 64), lambda i: (i, 0)),
            pl.BlockSpec((rb, 16), lambda i: (i, 0)),
            pl.BlockSpec((rb, 128), lambda i: (i, 0)),
            pl.BlockSpec((1, 128), lambda i: (0, 0)),
            pl.BlockSpec((128, 384), lambda i: (0, 0)),
            pl.BlockSpec((128, 384), lambda i: (0, 0)),
            pl.BlockSpec((1, 384), lambda i: (0, 0)),
            pl.BlockSpec((1, 384), lambda i: (0, 0)),
        ],
        out_specs=pl.BlockSpec((rb, 128), lambda i: (i, 0)),
        out_shape=jax.ShapeDtypeStruct((M, 128), _f32),
    )(P0, P1, D, xprev, bias.reshape(1, 128),
      wihT, whhT, bih.reshape(1, 384), bhh.reshape(1, 384))


_NC, _NS, _L = 2, 16, 16
_CH = 80


def _acc_split(accr):
    base = (accr // _NS) & ~7
    tile_rows = [base] * (_NS - 1) + [accr - base * (_NS - 1)]
    tile_base = [base * k for k in range(_NS)]

    def pieces_for(tr):
        out, off = [], 0
        while off < tr:
            sz = min(_CH, tr - off)
            out.append((off, sz))
            off += sz
        return out
    return tile_base, tile_rows, pieces_for


def _make_sc_edge(e_total, accr, has_eaq):
    mesh = plsc.VectorSubcoreMesh(core_axis_name="c", subcore_axis_name="s")
    epw = e_total // _NS
    nchunk = epw // _CH
    qw = 32 if has_eaq else 16
    tile_base, tile_rows, pieces_for = _acc_split(accr)

    out_type = (jax.ShapeDtypeStruct((_NC, accr, 64), _f32),
                jax.ShapeDtypeStruct((accr, qw), _f32))
    scratch = [
        pltpu.VMEM((NP,), _f32),
        pltpu.VMEM((NP,), _f32),
        pltpu.VMEM((_CH,), jnp.int32),
        pltpu.VMEM((_CH,), jnp.int32),
        pltpu.VMEM((_CH,), jnp.int32),
        pltpu.VMEM((_CH, 64), _f32),
        pltpu.VMEM((_CH, qw), _f32),
        pltpu.VMEM((_CH,), _f32),
        pltpu.VMEM_SHARED((accr, 64), _f32),
        pltpu.VMEM_SHARED((accr, qw), _f32),
        pltpu.SemaphoreType.DMA,
    ]

    def body(*refs):
        if has_eaq:
            (s_hbm, r_hbm, src_hbm, dst_hbm, t2_hbm, eaq_hbm,
             p_out, qd_out,
             s_v, r_v, srcb, dstb, idx2, rows, eab, alpha,
             accp, accq, sem) = refs
        else:
            (s_hbm, r_hbm, src_hbm, dst_hbm, t2_hbm,
             p_out, qd_out,
             s_v, r_v, srcb, dstb, idx2, rows, eab, alpha,
             accp, accq, sem) = refs
            eaq_hbm = None
        cid = lax.axis_index("c")
        sid = lax.axis_index("s")
        zero16 = jnp.zeros((16,), _f32)
        onehot = (lax.iota(jnp.int32, 16) == 0).astype(_f32)
        lane = lax.iota(jnp.int32, 16)

        def zrow(i, carry):
            for j in range(4):
                rows[i, pl.ds(16 * j, 16)] = zero16
            for j in range(qw // 16):
                eab[i, pl.ds(16 * j, 16)] = zero16
            return carry
        lax.fori_loop(0, _CH, zrow, 0)

        for k in range(_NS):
            @pl.when(sid == k)
            def _():
                for (off, sz) in pieces_for(tile_rows[k]):
                    r0 = tile_base[k] + off
                    pltpu.sync_copy(rows.at[pl.ds(0, sz)],
                                    accp.at[pl.ds(r0, sz)])
                    pltpu.sync_copy(eab.at[pl.ds(0, sz)],
                                    accq.at[pl.ds(r0, sz)])

        pltpu.sync_copy(s_hbm, s_v)
        pltpu.sync_copy(r_hbm, r_v)
        plsc.subcore_barrier()
        ebase = sid * epw
        coff = cid * NP

        def chunk(ci, carry):
            e0 = ebase + ci * _CH
            pltpu.sync_copy(src_hbm.at[pl.ds(e0, _CH)], srcb)
            pltpu.sync_copy(dst_hbm.at[pl.ds(e0, _CH)], dstb)
            if has_eaq:
                pltpu.sync_copy(eaq_hbm.at[pl.ds(e0, _CH)], eab)

            def mkidx(g, c2):
                idx2[pl.ds(g * 16, 16)] = srcb[pl.ds(g * 16, 16)] + coff
                return c2
            lax.fori_loop(0, _CH // 16, mkidx, 0)
            pltpu.async_copy(t2_hbm.at[idx2], rows, sem).wait()

            def grp(g, c2):
                si = srcb[pl.ds(g * 16, 16)]
                di = dstb[pl.ds(g * 16, 16)]
                sv = plsc.load_gather(s_v, [si])
                rv = plsc.load_gather(r_v, [di])
                z = sv + rv
                if has_eaq:
                    qv = plsc.load_gather(
                        eab, [g * 16 + lane,
                              jnp.full((16,), 17, jnp.int32)])
                    z = z + qv
                zl = jnp.where(z > 0, z, 0.2 * z)
                al = jnp.exp(zl - jnp.maximum(rv, 0.0))
                alpha[pl.ds(g * 16, 16)] = al
                return c2
            lax.fori_loop(0, _CH // 16, grp, 0)

            def scale(e, c2):
                av = plsc.load_gather(alpha, [jnp.full((16,), e, jnp.int32)])
                for j in range(4):
                    rows[e, pl.ds(16 * j, 16)] = (
                        rows[e, pl.ds(16 * j, 16)] * av)
                return c2
            lax.fori_loop(0, _CH, scale, 0)

            @pl.when(cid == 0)
            def _():
                def scq(e, c2):
                    av = plsc.load_gather(
                        alpha, [jnp.full((16,), e, jnp.int32)])
                    if has_eaq:
                        for j in range(2):
                            eab[e, pl.ds(16 * j, 16)] = (
                                eab[e, pl.ds(16 * j, 16)] * av)
                    else:
                        eab[e, pl.ds(0, 16)] = av * onehot
                    return c2
                lax.fori_loop(0, _CH, scq, 0)

            pltpu.sync_copy(rows, accp.at[dstb], add=True)

            @pl.when(cid == 0)
            def _():
                pltpu.sync_copy(eab, accq.at[dstb], add=True)
            return carry
        lax.fori_loop(0, nchunk, chunk, 0)

        plsc.subcore_barrier()

        for k in range(_NS):
            @pl.when(sid == k)
            def _():
                for (off, sz) in pieces_for(tile_rows[k]):
                    r0 = tile_base[k] + off
                    pltpu.sync_copy(accp.at[pl.ds(r0, sz)],
                                    p_out.at[cid, pl.ds(r0, sz)])

                    @pl.when(cid == 0)
                    def _():
                        pltpu.sync_copy(accq.at[pl.ds(r0, sz)],
                                        qd_out.at[pl.ds(r0, sz)])

    return pl.kernel(
        body, mesh=mesh,
        compiler_params=pltpu.CompilerParams(
            use_tc_tiling_on_sc=False, needs_layout_passes=False),
        out_type=out_type, scratch_types=scratch)


def _sc_aggregate(s, r, src, dst, t, accr, eaq=None):
    e_total = src.shape[0]
    t2 = jnp.concatenate([t[:, :64], t[:, 64:]], axis=0)
    k = _make_sc_edge(e_total, accr, eaq is not None)
    if eaq is not None:
        return k(s, r, src, dst, t2, eaq)
    return k(s, r, src, dst, t2)


def _combine_body(a_ref, b_ref, o_ref):
    o_ref[...] = jnp.concatenate([a_ref[...], b_ref[...]], axis=1)


def _combine(a, b):
    M, K = a.shape
    return pl.pallas_call(
        _combine_body,
        grid=(1,),
        in_specs=[pl.BlockSpec((M, K), lambda i: (0, 0)),
                  pl.BlockSpec((M, K), lambda i: (0, 0))],
        out_specs=pl.BlockSpec((M, 2 * K), lambda i: (0, 0)),
        out_shape=jax.ShapeDtypeStruct((M, 2 * K), _f32),
    )(a, b)


def kernel(x, edge_index, edge_attr, batch, params):
    p = params
    src = edge_index[0].astype(jnp.int32)
    dst = edge_index[1].astype(jnp.int32)

    xp = jnp.pad(x, ((0, NP - N), (0, 0)))
    x0 = _mm(xp, p['lin1_W'].T, p['lin1_b'], act='relu')

    Wl = p['gate_lin_l_W']
    WlxT, Wle = Wl[:, :H].T, Wl[:, H:]
    t = _mm(x0, WlxT, jnp.zeros((H,), _f32))
    s = _rowdots(t, p['gate_att_l'].reshape(1, 128))[0]
    vr = p['gate_lin_r_W'].T @ p['gate_att_r']
    r = _rowdots(x0, vr.reshape(1, 128))[0]

    wq = Wle.T @ p['gate_att_l']
    Mmat = jnp.zeros((128, 256), _f32)
    for j in range(8):
        Mmat = Mmat.at[16 * j:16 * j + 16, 32 * j:32 * j + 16].set(
            jnp.eye(16, dtype=_f32))
        Mmat = Mmat.at[16 * j:16 * j + 16, 32 * j + 17].set(wq)
    onesrow = jnp.zeros((1, 256), _f32)
    for j in range(8):
        onesrow = onesrow.at[0, 32 * j + 16].set(1.0)
    eaq = _build_eaq(edge_attr.reshape(E // 8, 128), Mmat, onesrow)
    eaq = eaq.reshape(E, 32)

    Pg2, QDg = _sc_aggregate(s, r, src, dst, t, N, eaq=eaq)
    x1 = _gate_update(Pg2[0], Pg2[1], QDg, x0[:N],
                      Wle.T.reshape(16, 128), p['gate_bias'],
                      p['agru0_Wih'].T, p['agru0_Whh'].T,
                      p['agru0_bih'], p['agru0_bhh'])
    x1 = jnp.pad(x1, ((0, NP - N), (0, 0)))

    hh = _mm(x1, p['aconv1_W'].T, jnp.zeros((H,), _f32))
    sr2 = _rowdots(hh, jnp.stack([p['aconv1_att_src'], p['aconv1_att_dst']]))
    s2, r2 = sr2[0], sr2[1]
    Pa2, Da = _sc_aggregate(s2, r2, src, dst, hh, N)
    x2 = _agg_update(Pa2[0], Pa2[1], Da, x1[:N],
                     p['aconv1_bias'],
                     p['agru1_Wih'].T, p['agru1_Whh'].T,
                     p['agru1_bih'], p['agru1_bhh'])
    x2 = jnp.pad(x2, ((0, NP - N), (0, 0)))

    BP = 512
    batch_p = jnp.pad(batch, (0, NP - N), constant_values=B).astype(jnp.int32)
    rowids = jnp.arange(NP, dtype=jnp.int32)
    xs = [x0, x1, x2]
    zeroN = jnp.zeros((NP,), _f32)
    P0m2, _ = _sc_aggregate(zeroN, zeroN, rowids, batch_p, x2, 528)
    out = _combine(P0m2[0, :BP], P0m2[1, :BP])
    for i in range(3):
        W = p['mconv%d_W' % i]
        hs = _mm(xs[i], W.T, jnp.zeros((H,), _f32))
        sm = _rowdots(hs, p['mconv%d_att_src' % i].reshape(1, 128))[0]
        hd = _mm(out, W.T, jnp.zeros((H,), _f32), rb=512)
        rm = _rowdots(hd, p['mconv%d_att_dst' % i].reshape(1, 128))[0]
        rm_p = jnp.pad(rm, (0, NP - BP))
        Pm2, Dm = _sc_aggregate(sm, rm_p, rowids, batch_p, hs, 528)
        out = _agg_update(Pm2[0, :BP], Pm2[1, :BP], Dm[:BP], out,
                          p['mconv%d_bias' % i],
                          p['mgru%d_Wih' % i].T, p['mgru%d_Whh' % i].T,
                          p['mgru%d_bih' % i], p['mgru%d_bhh' % i], rb=512)

    return _mm(out, p['lin2_W'].T, p['lin2_b'], rb=512)


# speedup vs baseline: 10.4691x; 1.5570x over previous
import functools

import jax
import jax.numpy as jnp
from jax import lax
from jax.experimental import pallas as pl
from jax.experimental.pallas import tpu as pltpu
from jax.experimental.pallas import tpu_sc as plsc

H = 128
ED = 16
B = 512
NP = 10240
N = 10000
E = 320000

_f32 = jnp.float32


def _mm_body(x_ref, w_ref, b_ref, o_ref, *, act):
    y = jnp.dot(x_ref[...], w_ref[...], preferred_element_type=_f32)
    y = y + b_ref[...]
    if act == 'relu':
        y = jnp.maximum(y, 0.0)
    o_ref[...] = y


def _mm(x, wT, b, act='none', rb=1024):
    M, K = x.shape
    Nout = wT.shape[1]
    grid = M // rb
    return pl.pallas_call(
        functools.partial(_mm_body, act=act),
        grid=(grid,),
        in_specs=[
            pl.BlockSpec((rb, K), lambda i: (i, 0)),
            pl.BlockSpec((K, Nout), lambda i: (0, 0)),
            pl.BlockSpec((1, Nout), lambda i: (0, 0)),
        ],
        out_specs=pl.BlockSpec((rb, Nout), lambda i: (i, 0)),
        out_shape=jax.ShapeDtypeStruct((M, Nout), _f32),
    )(x, wT, b.reshape(1, Nout))


def _rowdot_body(m_ref, v_ref, o_ref):
    blk = m_ref[0]
    o_ref[0] = lax.dot_general(
        v_ref[...], blk, (((1,), (1,)), ((), ())),
        preferred_element_type=_f32)


def _rowdots(mat, vecs):
    M = mat.shape[0]
    V = vecs.shape[0]
    G = M // 128
    mat3 = mat.reshape(G, 128, 128)
    out = pl.pallas_call(
        _rowdot_body,
        grid=(G,),
        in_specs=[
            pl.BlockSpec((1, 128, 128), lambda i: (i, 0, 0)),
            pl.BlockSpec((V, 128), lambda i: (0, 0)),
        ],
        out_specs=pl.BlockSpec((1, V, 128), lambda i: (i, 0, 0)),
        out_shape=jax.ShapeDtypeStruct((G, V, 128), _f32),
    )(mat3, vecs)
    return out.transpose(1, 0, 2).reshape(V, M)


def _eaq_body(ea_ref, m_ref, ones_ref, o_ref):
    o_ref[...] = jnp.dot(ea_ref[...], m_ref[...],
                         preferred_element_type=_f32) + ones_ref[...]


def _build_eaq(ea2, Mmat, onesrow, rb=2000):
    M = ea2.shape[0]
    return pl.pallas_call(
        _eaq_body,
        grid=(M // rb,),
        in_specs=[
            pl.BlockSpec((rb, 128), lambda i: (i, 0)),
            pl.BlockSpec((128, 256), lambda i: (0, 0)),
            pl.BlockSpec((1, 256), lambda i: (0, 0)),
        ],
        out_specs=pl.BlockSpec((rb, 256), lambda i: (i, 0)),
        out_shape=jax.ShapeDtypeStruct((M, 256), _f32),
    )(ea2, Mmat, onesrow)


def _sigmoid(x):
    return 1.0 / (1.0 + jnp.exp(-x))


def _gru_from(gi, gh, h):
    ir, iz, inn = gi[:, :H], gi[:, H:2 * H], gi[:, 2 * H:]
    hr, hz, hn = gh[:, :H], gh[:, H:2 * H], gh[:, 2 * H:]
    r = _sigmoid(ir + hr)
    z = _sigmoid(iz + hz)
    nn_ = jnp.tanh(inn + r * hn)
    return (1.0 - z) * nn_ + z * h


def _gate_update_body(p0_ref, p1_ref, qd_ref, x0_ref,
                      wle_ref, bias_ref, wih_ref, whh_ref, bih_ref, bhh_ref,
                      o_ref):
    qd = qd_ref[...]
    P = jnp.concatenate([p0_ref[...], p1_ref[...]], axis=1)
    den = qd[:, 16:17] + 1e-16
    h = (P + jnp.dot(qd[:, :16], wle_ref[...],
                     preferred_element_type=_f32)) / den + bias_ref[...]
    h = jnp.where(h > 0, h, jnp.exp(jnp.minimum(h, 0.0)) - 1.0)
    x0 = x0_ref[...]
    gi = jnp.dot(h, wih_ref[...], preferred_element_type=_f32) + bih_ref[...]
    gh = jnp.dot(x0, whh_ref[...], preferred_element_type=_f32) + bhh_ref[...]
    o_ref[...] = _gru_from(gi, gh, x0)


def _gate_update(P0, P1, QD, x0, wleT, bias, wihT, whhT, bih, bhh,
                 rb=1000):
    M = x0.shape[0]
    return pl.pallas_call(
        _gate_update_body,
        grid=(M // rb,),
        in_specs=[
            pl.BlockSpec((rb, 64), lambda i: (i, 0)),
            pl.BlockSpec((rb, 64), lambda i: (i, 0)),
            pl.BlockSpec((rb, 32), lambda i: (i, 0)),
            pl.BlockSpec((rb, 128), lambda i: (i, 0)),
            pl.BlockSpec((16, 128), lambda i: (0, 0)),
            pl.BlockSpec((1, 128), lambda i: (0, 0)),
            pl.BlockSpec((128, 384), lambda i: (0, 0)),
            pl.BlockSpec((128, 384), lambda i: (0, 0)),
            pl.BlockSpec((1, 384), lambda i: (0, 0)),
            pl.BlockSpec((1, 384), lambda i: (0, 0)),
        ],
        out_specs=pl.BlockSpec((rb, 128), lambda i: (i, 0)),
        out_shape=jax.ShapeDtypeStruct((M, 128), _f32),
    )(P0, P1, QD, x0, wleT, bias.reshape(1, 128),
      wihT, whhT, bih.reshape(1, 384), bhh.reshape(1, 384))


def _agg_update_body(p0_ref, p1_ref, d_ref, xp_ref,
                     bias_ref, wih_ref, whh_ref, bih_ref, bhh_ref, o_ref):
    den = d_ref[:, 0:1] + 1e-16
    P = jnp.concatenate([p0_ref[...], p1_ref[...]], axis=1)
    h = P / den + bias_ref[...]
    h = jnp.where(h > 0, h, jnp.exp(jnp.minimum(h, 0.0)) - 1.0)
    xp = xp_ref[...]
    gi = jnp.dot(h, wih_ref[...], preferred_element_type=_f32) + bih_ref[...]
    gh = jnp.dot(xp, whh_ref[...], preferred_element_type=_f32) + bhh_ref[...]
    o_ref[...] = _gru_from(gi, gh, xp)


def _agg_update(P0, P1, D, xprev, bias, wihT, whhT, bih, bhh, rb=1000):
    M = xprev.shape[0]
    return pl.pallas_call(
        _agg_update_body,
        grid=(M // rb,),
        in_specs=[
            pl.BlockSpec((rb, 64), lambda i: (i, 0)),
            pl.BlockSpec((rb, 64), lambda i: (i, 0)),
            pl.BlockSpec((rb, 16), lambda i: (i, 0)),
            pl.BlockSpec((rb, 128), lambda i: (i, 0)),
            pl.BlockSpec((1, 128), lambda i: (0, 0)),
            pl.BlockSpec((128, 384), lambda i: (0, 0)),
            pl.BlockSpec((128, 384), lambda i: (0, 0)),
            pl.BlockSpec((1, 384), lambda i: (0, 0)),
            pl.BlockSpec((1, 384), lambda i: (0, 0)),
        ],
        out_specs=pl.BlockSpec((rb, 128), lambda i: (i, 0)),
        out_shape=jax.ShapeDtypeStruct((M, 128), _f32),
    )(P0, P1, D, xprev, bias.reshape(1, 128),
      wihT, whhT, bih.reshape(1, 384), bhh.reshape(1, 384))


_NC, _NS, _L = 2, 16, 16
_CH = 80


def _acc_split(accr):
    base = (accr // _NS) & ~7
    tile_rows = [base] * (_NS - 1) + [accr - base * (_NS - 1)]
    tile_base = [base * k for k in range(_NS)]

    def pieces_for(tr):
        out, off = [], 0
        while off < tr:
            sz = min(_CH, tr - off)
            out.append((off, sz))
            off += sz
        return out
    return tile_base, tile_rows, pieces_for


def _make_sc_edge(e_total, accr, has_eaq):
    mesh = plsc.VectorSubcoreMesh(core_axis_name="c", subcore_axis_name="s")
    m = e_total // (_CH * _NS)
    S = min(25, m)
    nsup = m // S
    qw = 32 if has_eaq else 16
    tile_base, tile_rows, pieces_for = _acc_split(accr)

    out_type = (jax.ShapeDtypeStruct((_NC, accr, 64), _f32),
                jax.ShapeDtypeStruct((accr, qw), _f32))
    scratch = [
        pltpu.VMEM((NP,), _f32),
        pltpu.VMEM((NP,), _f32),
        pltpu.VMEM((S, _CH), jnp.int32),
        pltpu.VMEM((S, _CH), jnp.int32),
        pltpu.VMEM((S, _CH), jnp.int32),
        pltpu.VMEM((_CH, 64), _f32),
        pltpu.VMEM((_CH, 64), _f32),
        pltpu.VMEM((_CH, qw), _f32),
        pltpu.VMEM((_CH, qw), _f32),
        pltpu.VMEM((_CH,), _f32),
        pltpu.SemaphoreType.DMA,
        pltpu.SemaphoreType.DMA,
        pltpu.SemaphoreType.DMA,
        pltpu.SemaphoreType.DMA,
        pltpu.SemaphoreType.DMA,
        pltpu.SemaphoreType.DMA,
        pltpu.SemaphoreType.DMA,
        pltpu.SemaphoreType.DMA,
        pltpu.VMEM_SHARED((accr, 64), _f32),
        pltpu.VMEM_SHARED((accr, qw), _f32),
    ]

    def body(*refs):
        if has_eaq:
            (s_hbm, r_hbm, src_hbm, dst_hbm, t2_hbm, eaq_hbm,
             p_out, qd_out, s_v, r_v, srcb, dstb, idx2,
             rows0, rows1, eab0, eab1, alpha,
             gsem0, gsem1, esem0, esem1, psem0, psem1, qsem0, qsem1,
             accp, accq) = refs
        else:
            (s_hbm, r_hbm, src_hbm, dst_hbm, t2_hbm,
             p_out, qd_out, s_v, r_v, srcb, dstb, idx2,
             rows0, rows1, eab0, eab1, alpha,
             gsem0, gsem1, esem0, esem1, psem0, psem1, qsem0, qsem1,
             accp, accq) = refs
            eaq_hbm = None
        rows_ = (rows0, rows1)
        eab_ = (eab0, eab1)
        gsem = (gsem0, gsem1)
        esem = (esem0, esem1)
        psem = (psem0, psem1)
        qsem = (qsem0, qsem1)
        cid = lax.axis_index("c")
        sid = lax.axis_index("s")
        zero16 = jnp.zeros((16,), _f32)
        onehot = (lax.iota(jnp.int32, 16) == 0).astype(_f32)
        lane = lax.iota(jnp.int32, 16)

        def zrow(i, carry):
            for j in range(4):
                rows0[i, pl.ds(16 * j, 16)] = zero16
            for j in range(qw // 16):
                eab0[i, pl.ds(16 * j, 16)] = zero16
            return carry
        lax.fori_loop(0, _CH, zrow, 0)

        for k in range(_NS):
            @pl.when(sid == k)
            def _():
                for (off, sz) in pieces_for(tile_rows[k]):
                    r0 = tile_base[k] + off
                    pltpu.sync_copy(rows0.at[pl.ds(0, sz)],
                                    accp.at[pl.ds(r0, sz)])
                    pltpu.sync_copy(eab0.at[pl.ds(0, sz)],
                                    accq.at[pl.ds(r0, sz)])

        pltpu.sync_copy(s_hbm, s_v)
        pltpu.sync_copy(r_hbm, r_v)
        plsc.subcore_barrier()
        coff = cid * NP

        def do_grp(eabk, ci):
            def grp(g, c2):
                si = srcb[ci, pl.ds(g * 16, 16)]
                di = dstb[ci, pl.ds(g * 16, 16)]
                sv = plsc.load_gather(s_v, [si])
                rv = plsc.load_gather(r_v, [di])
                z = sv + rv
                if has_eaq:
                    qv = plsc.load_gather(
                        eabk, [g * 16 + lane,
                               jnp.full((16,), 17, jnp.int32)])
                    z = z + qv
                zl = jnp.where(z > 0, z, 0.2 * z)
                al = jnp.exp(zl - jnp.maximum(rv, 0.0))
                alpha[pl.ds(g * 16, 16)] = al
                return c2
            lax.fori_loop(0, _CH // 16, grp, 0)

        def do_scale(rowsk, eabk):
            def scale(e, c2):
                av = plsc.load_gather(alpha, [jnp.full((16,), e, jnp.int32)])
                for j in range(4):
                    rowsk[e, pl.ds(16 * j, 16)] = (
                        rowsk[e, pl.ds(16 * j, 16)] * av)
                return c2
            lax.fori_loop(0, _CH, scale, 0)

            @pl.when(cid == 0)
            def _():
                def scq(e, c2):
                    av = plsc.load_gather(
                        alpha, [jnp.full((16,), e, jnp.int32)])
                    if has_eaq:
                        for j in range(2):
                            eabk[e, pl.ds(16 * j, 16)] = (
                                eabk[e, pl.ds(16 * j, 16)] * av)
                    else:
                        eabk[e, pl.ds(0, 16)] = av * onehot
                    return c2
                lax.fori_loop(0, _CH, scq, 0)

        def super_block(sj, carry):
            row0 = sid * m + sj * S
            pltpu.sync_copy(src_hbm.at[pl.ds(row0, S)], srcb)
            pltpu.sync_copy(dst_hbm.at[pl.ds(row0, S)], dstb)

            def mkidx(i, c2):
                for g in range(_CH // 16):
                    idx2[i, pl.ds(g * 16, 16)] = (
                        srcb[i, pl.ds(g * 16, 16)] + coff)
                return c2
            lax.fori_loop(0, S, mkidx, 0)

            gh = {}
            eh = {}
            if has_eaq:
                eh[0] = pltpu.async_copy(
                    eaq_hbm.at[pl.ds(row0 * _CH, _CH)], eab_[0], esem[0])
            gh[0] = pltpu.async_copy(t2_hbm.at[idx2.at[0]], rows_[0], gsem[0])

            ph = {}
            for c in range(S):
                k = c % 2
                nk = 1 - k
                if c + 1 < S:
                    if c + 1 >= 2:
                        ph.pop(nk).wait()
                    if has_eaq:
                        eh[c + 1] = pltpu.async_copy(
                            eaq_hbm.at[pl.ds((row0 + c + 1) * _CH, _CH)],
                            eab_[nk], esem[nk])
                    gh[c + 1] = pltpu.async_copy(
                        t2_hbm.at[idx2.at[c + 1]], rows_[nk], gsem[nk])
                gh.pop(c).wait()
                if has_eaq:
                    eh.pop(c).wait()
                do_grp(eab_[k], c)
                do_scale(rows_[k], eab_[k])
                ph[k] = pltpu.async_copy(
                    rows_[k], accp.at[dstb.at[c]], psem[k], add=True)

                @pl.when(cid == 0)
                def _():
                    pltpu.async_copy(
                        eab_[k], accq.at[dstb.at[c]], qsem[k],
                        add=True).wait()
            for k in list(ph):
                ph.pop(k).wait()
            return carry
        lax.fori_loop(0, nsup, super_block, 0)

        plsc.subcore_barrier()

        for k in range(_NS):
            @pl.when(sid == k)
            def _():
                for (off, sz) in pieces_for(tile_rows[k]):
                    r0 = tile_base[k] + off
                    pltpu.sync_copy(accp.at[pl.ds(r0, sz)],
                                    p_out.at[cid, pl.ds(r0, sz)])

                    @pl.when(cid == 0)
                    def _():
                        pltpu.sync_copy(accq.at[pl.ds(r0, sz)],
                                        qd_out.at[pl.ds(r0, sz)])

    return pl.kernel(
        body, mesh=mesh,
        compiler_params=pltpu.CompilerParams(
            use_tc_tiling_on_sc=False, needs_layout_passes=False),
        out_type=out_type, scratch_types=scratch)


def _sc_aggregate(s, r, src, dst, t, accr, eaq=None):
    e_total = src.shape[0]
    t2 = jnp.concatenate([t[:, :64], t[:, 64:]], axis=0)
    src2 = src.reshape(e_total // _CH, _CH)
    dst2 = dst.reshape(e_total // _CH, _CH)
    k = _make_sc_edge(e_total, accr, eaq is not None)
    if eaq is not None:
        return k(s, r, src2, dst2, t2, eaq)
    return k(s, r, src2, dst2, t2)


def _combine_body(a_ref, b_ref, o_ref):
    o_ref[...] = jnp.concatenate([a_ref[...], b_ref[...]], axis=1)


def _combine(a, b):
    M, K = a.shape
    return pl.pallas_call(
        _combine_body,
        grid=(1,),
        in_specs=[pl.BlockSpec((M, K), lambda i: (0, 0)),
                  pl.BlockSpec((M, K), lambda i: (0, 0))],
        out_specs=pl.BlockSpec((M, 2 * K), lambda i: (0, 0)),
        out_shape=jax.ShapeDtypeStruct((M, 2 * K), _f32),
    )(a, b)


def kernel(x, edge_index, edge_attr, batch, params):
    p = params
    src = edge_index[0].astype(jnp.int32)
    dst = edge_index[1].astype(jnp.int32)

    xp = jnp.pad(x, ((0, NP - N), (0, 0)))
    x0 = _mm(xp, p['lin1_W'].T, p['lin1_b'], act='relu')

    Wl = p['gate_lin_l_W']
    WlxT, Wle = Wl[:, :H].T, Wl[:, H:]
    t = _mm(x0, WlxT, jnp.zeros((H,), _f32))
    s = _rowdots(t, p['gate_att_l'].reshape(1, 128))[0]
    vr = p['gate_lin_r_W'].T @ p['gate_att_r']
    r = _rowdots(x0, vr.reshape(1, 128))[0]

    wq = Wle.T @ p['gate_att_l']
    Mmat = jnp.zeros((128, 256), _f32)
    for j in range(8):
        Mmat = Mmat.at[16 * j:16 * j + 16, 32 * j:32 * j + 16].set(
            jnp.eye(16, dtype=_f32))
        Mmat = Mmat.at[16 * j:16 * j + 16, 32 * j + 17].set(wq)
    onesrow = jnp.zeros((1, 256), _f32)
    for j in range(8):
        onesrow = onesrow.at[0, 32 * j + 16].set(1.0)
    eaq = _build_eaq(edge_attr.reshape(E // 8, 128), Mmat, onesrow)
    eaq = eaq.reshape(E, 32)

    Pg2, QDg = _sc_aggregate(s, r, src, dst, t, N, eaq=eaq)
    x1 = _gate_update(Pg2[0], Pg2[1], QDg, x0[:N],
                      Wle.T.reshape(16, 128), p['gate_bias'],
                      p['agru0_Wih'].T, p['agru0_Whh'].T,
                      p['agru0_bih'], p['agru0_bhh'])
    x1 = jnp.pad(x1, ((0, NP - N), (0, 0)))

    hh = _mm(x1, p['aconv1_W'].T, jnp.zeros((H,), _f32))
    sr2 = _rowdots(hh, jnp.stack([p['aconv1_att_src'], p['aconv1_att_dst']]))
    s2, r2 = sr2[0], sr2[1]
    Pa2, Da = _sc_aggregate(s2, r2, src, dst, hh, N)
    x2 = _agg_update(Pa2[0], Pa2[1], Da, x1[:N],
                     p['aconv1_bias'],
                     p['agru1_Wih'].T, p['agru1_Whh'].T,
                     p['agru1_bih'], p['agru1_bhh'])
    x2 = jnp.pad(x2, ((0, NP - N), (0, 0)))

    BP = 512
    batch_p = jnp.pad(batch, (0, NP - N), constant_values=B).astype(jnp.int32)
    rowids = jnp.arange(NP, dtype=jnp.int32)
    xs = [x0, x1, x2]
    zeroN = jnp.zeros((NP,), _f32)
    P0m2, _ = _sc_aggregate(zeroN, zeroN, rowids, batch_p, x2, 528)
    out = _combine(P0m2[0, :BP], P0m2[1, :BP])
    for i in range(3):
        W = p['mconv%d_W' % i]
        hs = _mm(xs[i], W.T, jnp.zeros((H,), _f32))
        sm = _rowdots(hs, p['mconv%d_att_src' % i].reshape(1, 128))[0]
        hd = _mm(out, W.T, jnp.zeros((H,), _f32), rb=512)
        rm = _rowdots(hd, p['mconv%d_att_dst' % i].reshape(1, 128))[0]
        rm_p = jnp.pad(rm, (0, NP - BP))
        Pm2, Dm = _sc_aggregate(sm, rm_p, rowids, batch_p, hs, 528)
        out = _agg_update(Pm2[0, :BP], Pm2[1, :BP], Dm[:BP], out,
                          p['mconv%d_bias' % i],
                          p['mgru%d_Wih' % i].T, p['mgru%d_Whh' % i].T,
                          p['mgru%d_bih' % i], p['mgru%d_bhh' % i], rb=512)

    return _mm(out, p['lin2_W'].T, p['lin2_b'], rb=512)


# speedup vs baseline: 12.4037x; 1.1848x over previous
import functools

import jax
import jax.numpy as jnp
from jax import lax
from jax.experimental import pallas as pl
from jax.experimental.pallas import tpu as pltpu
from jax.experimental.pallas import tpu_sc as plsc

H = 128
ED = 16
B = 512
NP = 10240
N = 10000
E = 320000

_f32 = jnp.float32


def _mm_body(x_ref, w_ref, b_ref, o_ref, *, act):
    y = jnp.dot(x_ref[...], w_ref[...], preferred_element_type=_f32)
    y = y + b_ref[...]
    if act == 'relu':
        y = jnp.maximum(y, 0.0)
    o_ref[...] = y


def _mm(x, wT, b, act='none', rb=1024):
    M, K = x.shape
    Nout = wT.shape[1]
    grid = M // rb
    return pl.pallas_call(
        functools.partial(_mm_body, act=act),
        grid=(grid,),
        in_specs=[
            pl.BlockSpec((rb, K), lambda i: (i, 0)),
            pl.BlockSpec((K, Nout), lambda i: (0, 0)),
            pl.BlockSpec((1, Nout), lambda i: (0, 0)),
        ],
        out_specs=pl.BlockSpec((rb, Nout), lambda i: (i, 0)),
        out_shape=jax.ShapeDtypeStruct((M, Nout), _f32),
    )(x, wT, b.reshape(1, Nout))


def _rowdot_body(m_ref, v_ref, o_ref):
    blk = m_ref[0]
    o_ref[0] = lax.dot_general(
        v_ref[...], blk, (((1,), (1,)), ((), ())),
        preferred_element_type=_f32)


def _rowdots(mat, vecs):
    M = mat.shape[0]
    V = vecs.shape[0]
    G = M // 128
    mat3 = mat.reshape(G, 128, 128)
    out = pl.pallas_call(
        _rowdot_body,
        grid=(G,),
        in_specs=[
            pl.BlockSpec((1, 128, 128), lambda i: (i, 0, 0)),
            pl.BlockSpec((V, 128), lambda i: (0, 0)),
        ],
        out_specs=pl.BlockSpec((1, V, 128), lambda i: (i, 0, 0)),
        out_shape=jax.ShapeDtypeStruct((G, V, 128), _f32),
    )(mat3, vecs)
    return out.transpose(1, 0, 2).reshape(V, M)


def _eaq_body(ea_ref, m_ref, ones_ref, o_ref):
    o_ref[...] = jnp.dot(ea_ref[...], m_ref[...],
                         preferred_element_type=_f32) + ones_ref[...]


def _build_eaq(ea2, Mmat, onesrow, rb=2000):
    M = ea2.shape[0]
    return pl.pallas_call(
        _eaq_body,
        grid=(M // rb,),
        in_specs=[
            pl.BlockSpec((rb, 128), lambda i: (i, 0)),
            pl.BlockSpec((128, 256), lambda i: (0, 0)),
            pl.BlockSpec((1, 256), lambda i: (0, 0)),
        ],
        out_specs=pl.BlockSpec((rb, 256), lambda i: (i, 0)),
        out_shape=jax.ShapeDtypeStruct((M, 256), _f32),
    )(ea2, Mmat, onesrow)


def _sigmoid(x):
    return 1.0 / (1.0 + jnp.exp(-x))


def _gru_from(gi, gh, h):
    ir, iz, inn = gi[:, :H], gi[:, H:2 * H], gi[:, 2 * H:]
    hr, hz, hn = gh[:, :H], gh[:, H:2 * H], gh[:, 2 * H:]
    r = _sigmoid(ir + hr)
    z = _sigmoid(iz + hz)
    nn_ = jnp.tanh(inn + r * hn)
    return (1.0 - z) * nn_ + z * h


def _gate_update_body(p0_ref, p1_ref, qd0_ref, qd1_ref, x0_ref,
                      wle_ref, bias_ref, wih_ref, whh_ref, bih_ref, bhh_ref,
                      o_ref):
    qd = qd0_ref[...] + qd1_ref[...]
    P = jnp.concatenate([p0_ref[...], p1_ref[...]], axis=1)
    den = qd[:, 16:17] + 1e-16
    h = (P + jnp.dot(qd[:, :16], wle_ref[...],
                     preferred_element_type=_f32)) / den + bias_ref[...]
    h = jnp.where(h > 0, h, jnp.exp(jnp.minimum(h, 0.0)) - 1.0)
    x0 = x0_ref[...]
    gi = jnp.dot(h, wih_ref[...], preferred_element_type=_f32) + bih_ref[...]
    gh = jnp.dot(x0, whh_ref[...], preferred_element_type=_f32) + bhh_ref[...]
    o_ref[...] = _gru_from(gi, gh, x0)


def _gate_update(P0, P1, QD0, QD1, x0, wleT, bias, wihT, whhT, bih, bhh,
                 rb=1000):
    M = x0.shape[0]
    return pl.pallas_call(
        _gate_update_body,
        grid=(M // rb,),
        in_specs=[
            pl.BlockSpec((rb, 64), lambda i: (i, 0)),
            pl.BlockSpec((rb, 64), lambda i: (i, 0)),
            pl.BlockSpec((rb, 32), lambda i: (i, 0)),
            pl.BlockSpec((rb, 32), lambda i: (i, 0)),
            pl.BlockSpec((rb, 128), lambda i: (i, 0)),
            pl.BlockSpec((16, 128), lambda i: (0, 0)),
            pl.BlockSpec((1, 128), lambda i: (0, 0)),
            pl.BlockSpec((128, 384), lambda i: (0, 0)),
            pl.BlockSpec((128, 384), lambda i: (0, 0)),
            pl.BlockSpec((1, 384), lambda i: (0, 0)),
            pl.BlockSpec((1, 384), lambda i: (0, 0)),
        ],
        out_specs=pl.BlockSpec((rb, 128), lambda i: (i, 0)),
        out_shape=jax.ShapeDtypeStruct((M, 128), _f32),
    )(P0, P1, QD0, QD1, x0, wleT, bias.reshape(1, 128),
      wihT, whhT, bih.reshape(1, 384), bhh.reshape(1, 384))


def _agg_update_body(p0_ref, p1_ref, d0_ref, d1_ref, xp_ref,
                     bias_ref, wih_ref, whh_ref, bih_ref, bhh_ref, o_ref):
    den = d0_ref[:, 0:1] + d1_ref[:, 0:1] + 1e-16
    P = jnp.concatenate([p0_ref[...], p1_ref[...]], axis=1)
    h = P / den + bias_ref[...]
    h = jnp.where(h > 0, h, jnp.exp(jnp.minimum(h, 0.0)) - 1.0)
    xp = xp_ref[...]
    gi = jnp.dot(h, wih_ref[...], preferred_element_type=_f32) + bih_ref[...]
    gh = jnp.dot(xp, whh_ref[...], preferred_element_type=_f32) + bhh_ref[...]
    o_ref[...] = _gru_from(gi, gh, xp)


def _agg_update(P0, P1, D0, D1, xprev, bias, wihT, whhT, bih, bhh, rb=1000):
    M = xprev.shape[0]
    return pl.pallas_call(
        _agg_update_body,
        grid=(M // rb,),
        in_specs=[
            pl.BlockSpec((rb, 64), lambda i: (i, 0)),
            pl.BlockSpec((rb, 64), lambda i: (i, 0)),
            pl.BlockSpec((rb, 16), lambda i: (i, 0)),
            pl.BlockSpec((rb, 16), lambda i: (i, 0)),
            pl.BlockSpec((rb, 128), lambda i: (i, 0)),
            pl.BlockSpec((1, 128), lambda i: (0, 0)),
            pl.BlockSpec((128, 384), lambda i: (0, 0)),
            pl.BlockSpec((128, 384), lambda i: (0, 0)),
            pl.BlockSpec((1, 384), lambda i: (0, 0)),
            pl.BlockSpec((1, 384), lambda i: (0, 0)),
        ],
        out_specs=pl.BlockSpec((rb, 128), lambda i: (i, 0)),
        out_shape=jax.ShapeDtypeStruct((M, 128), _f32),
    )(P0, P1, D0, D1, xprev, bias.reshape(1, 128),
      wihT, whhT, bih.reshape(1, 384), bhh.reshape(1, 384))


_NC, _NS, _L = 2, 16, 16
_CH = 80


def _acc_split(accr):
    base = (accr // _NS) & ~7
    tile_rows = [base] * (_NS - 1) + [accr - base * (_NS - 1)]
    tile_base = [base * k for k in range(_NS)]

    def pieces_for(tr):
        out, off = [], 0
        while off < tr:
            sz = min(_CH, tr - off)
            out.append((off, sz))
            off += sz
        return out
    return tile_base, tile_rows, pieces_for


def _make_sc_edge(e_total, accr, has_eaq):
    mesh = plsc.VectorSubcoreMesh(core_axis_name="c", subcore_axis_name="s")
    m = e_total // (_CH * _NS)
    S = min(25, m)
    nsup = m // S
    qw = 32 if has_eaq else 16
    tile_base, tile_rows, pieces_for = _acc_split(accr)

    out_type = (jax.ShapeDtypeStruct((_NC, accr, 64), _f32),
                jax.ShapeDtypeStruct((_NC, accr, qw), _f32))
    scratch = [
        pltpu.VMEM((NP,), _f32),
        pltpu.VMEM((NP,), _f32),
        pltpu.VMEM((S, _CH), jnp.int32),
        pltpu.VMEM((S, _CH), jnp.int32),
        pltpu.VMEM((S, _CH), jnp.int32),
        pltpu.VMEM((_CH, 64), _f32),
        pltpu.VMEM((_CH, 64), _f32),
        pltpu.VMEM((_CH, qw), _f32),
        pltpu.VMEM((_CH, qw), _f32),
        pltpu.VMEM((_CH,), _f32),
        pltpu.SemaphoreType.DMA,
        pltpu.SemaphoreType.DMA,
        pltpu.SemaphoreType.DMA,
        pltpu.SemaphoreType.DMA,
        pltpu.SemaphoreType.DMA,
        pltpu.SemaphoreType.DMA,
        pltpu.SemaphoreType.DMA,
        pltpu.SemaphoreType.DMA,
        pltpu.VMEM_SHARED((accr, 64), _f32),
        pltpu.VMEM_SHARED((accr, qw), _f32),
    ]

    def body(*refs):
        if has_eaq:
            (s_hbm, r_hbm, src_hbm, dst_hbm, t2_hbm, eaq_hbm,
             p_out, qd_out, s_v, r_v, srcb, dstb, idx2,
             rows0, rows1, eab0, eab1, alpha,
             gsem0, gsem1, esem0, esem1, psem0, psem1, qsem0, qsem1,
             accp, accq) = refs
        else:
            (s_hbm, r_hbm, src_hbm, dst_hbm, t2_hbm,
             p_out, qd_out, s_v, r_v, srcb, dstb, idx2,
             rows0, rows1, eab0, eab1, alpha,
             gsem0, gsem1, esem0, esem1, psem0, psem1, qsem0, qsem1,
             accp, accq) = refs
            eaq_hbm = None
        rows_ = (rows0, rows1)
        eab_ = (eab0, eab1)
        gsem = (gsem0, gsem1)
        esem = (esem0, esem1)
        psem = (psem0, psem1)
        qsem = (qsem0, qsem1)
        cid = lax.axis_index("c")
        sid = lax.axis_index("s")
        zero16 = jnp.zeros((16,), _f32)
        onehot = (lax.iota(jnp.int32, 16) == 0).astype(_f32)
        lane = lax.iota(jnp.int32, 16)

        def zrow(i, carry):
            for j in range(4):
                rows0[i, pl.ds(16 * j, 16)] = zero16
            for j in range(qw // 16):
                eab0[i, pl.ds(16 * j, 16)] = zero16
            return carry
        lax.fori_loop(0, _CH, zrow, 0)

        for k in range(_NS):
            @pl.when(sid == k)
            def _():
                for (off, sz) in pieces_for(tile_rows[k]):
                    r0 = tile_base[k] + off
                    pltpu.sync_copy(rows0.at[pl.ds(0, sz)],
                                    accp.at[pl.ds(r0, sz)])
                    pltpu.sync_copy(eab0.at[pl.ds(0, sz)],
                                    accq.at[pl.ds(r0, sz)])

        pltpu.sync_copy(s_hbm, s_v)
        pltpu.sync_copy(r_hbm, r_v)
        plsc.subcore_barrier()

        def do_grp(eabk, ci):
            def grp(g, c2):
                si = srcb[ci, pl.ds(g * 16, 16)]
                di = dstb[ci, pl.ds(g * 16, 16)]
                sv = plsc.load_gather(s_v, [si])
                rv = plsc.load_gather(r_v, [di])
                z = sv + rv
                if has_eaq:
                    qv = plsc.load_gather(
                        eabk, [g * 16 + lane,
                               jnp.full((16,), 17, jnp.int32)])
                    z = z + qv
                zl = jnp.where(z > 0, z, 0.2 * z)
                al = jnp.exp(zl - jnp.maximum(rv, 0.0))
                alpha[pl.ds(g * 16, 16)] = al
                return c2
            lax.fori_loop(0, _CH // 16, grp, 0)

        def do_scale(rowsk, eabk, qduty):
            def scale(e, c2):
                av = plsc.load_gather(alpha, [jnp.full((16,), e, jnp.int32)])
                for j in range(4):
                    rowsk[e, pl.ds(16 * j, 16)] = (
                        rowsk[e, pl.ds(16 * j, 16)] * av)
                return c2
            lax.fori_loop(0, _CH, scale, 0)

            @pl.when(qduty)
            def _():
                def scq(e, c2):
                    av = plsc.load_gather(
                        alpha, [jnp.full((16,), e, jnp.int32)])
                    if has_eaq:
                        for j in range(2):
                            eabk[e, pl.ds(16 * j, 16)] = (
                                eabk[e, pl.ds(16 * j, 16)] * av)
                    else:
                        eabk[e, pl.ds(0, 16)] = av * onehot
                    return c2
                lax.fori_loop(0, _CH, scq, 0)

        def super_block(sj, carry):
            qduty = cid == lax.rem(sj, 2)
            row0 = sid * m + sj * S
            pltpu.sync_copy(src_hbm.at[pl.ds(row0, S)], srcb)
            pltpu.sync_copy(dst_hbm.at[pl.ds(row0, S)], dstb)

            def mkidx(i, c2):
                for g in range(_CH // 16):
                    idx2[i, pl.ds(g * 16, 16)] = (
                        srcb[i, pl.ds(g * 16, 16)] * 2 + cid)
                return c2
            lax.fori_loop(0, S, mkidx, 0)

            gh = {}
            eh = {}
            if has_eaq:
                eh[0] = pltpu.async_copy(
                    eaq_hbm.at[pl.ds(row0 * _CH, _CH)], eab_[0], esem[0])
            gh[0] = pltpu.async_copy(t2_hbm.at[idx2.at[0]], rows_[0], gsem[0])

            ph = {}
            for c in range(S):
                k = c % 2
                nk = 1 - k
                if c + 1 < S:
                    if c + 1 >= 2:
                        ph.pop(nk).wait()
                    if has_eaq:
                        eh[c + 1] = pltpu.async_copy(
                            eaq_hbm.at[pl.ds((row0 + c + 1) * _CH, _CH)],
                            eab_[nk], esem[nk])
                    gh[c + 1] = pltpu.async_copy(
                        t2_hbm.at[idx2.at[c + 1]], rows_[nk], gsem[nk])
                gh.pop(c).wait()
                if has_eaq:
                    eh.pop(c).wait()
                do_grp(eab_[k], c)
                do_scale(rows_[k], eab_[k], qduty)
                ph[k] = pltpu.async_copy(
                    rows_[k], accp.at[dstb.at[c]], psem[k], add=True)

                @pl.when(qduty)
                def _():
                    pltpu.async_copy(
                        eab_[k], accq.at[dstb.at[c]], qsem[k],
                        add=True).wait()
            for k in list(ph):
                ph.pop(k).wait()
            return carry
        lax.fori_loop(0, nsup, super_block, 0)

        plsc.subcore_barrier()

        for k in range(_NS):
            @pl.when(sid == k)
            def _():
                for (off, sz) in pieces_for(tile_rows[k]):
                    r0 = tile_base[k] + off
                    pltpu.sync_copy(accp.at[pl.ds(r0, sz)],
                                    p_out.at[cid, pl.ds(r0, sz)])
                    pltpu.sync_copy(accq.at[pl.ds(r0, sz)],
                                    qd_out.at[cid, pl.ds(r0, sz)])

    return pl.kernel(
        body, mesh=mesh,
        compiler_params=pltpu.CompilerParams(
            use_tc_tiling_on_sc=False, needs_layout_passes=False),
        out_type=out_type, scratch_types=scratch)


def _sc_aggregate(s, r, src, dst, t, accr, eaq=None):
    e_total = src.shape[0]
    t2 = t.reshape(2 * t.shape[0], 64)
    src2 = src.reshape(e_total // _CH, _CH)
    dst2 = dst.reshape(e_total // _CH, _CH)
    k = _make_sc_edge(e_total, accr, eaq is not None)
    if eaq is not None:
        return k(s, r, src2, dst2, t2, eaq)
    return k(s, r, src2, dst2, t2)


def _combine_body(a_ref, b_ref, o_ref):
    o_ref[...] = jnp.concatenate([a_ref[...], b_ref[...]], axis=1)


def _combine(a, b):
    M, K = a.shape
    return pl.pallas_call(
        _combine_body,
        grid=(1,),
        in_specs=[pl.BlockSpec((M, K), lambda i: (0, 0)),
                  pl.BlockSpec((M, K), lambda i: (0, 0))],
        out_specs=pl.BlockSpec((M, 2 * K), lambda i: (0, 0)),
        out_shape=jax.ShapeDtypeStruct((M, 2 * K), _f32),
    )(a, b)


def kernel(x, edge_index, edge_attr, batch, params):
    p = params
    src = edge_index[0].astype(jnp.int32)
    dst = edge_index[1].astype(jnp.int32)

    xp = jnp.pad(x, ((0, NP - N), (0, 0)))
    x0 = _mm(xp, p['lin1_W'].T, p['lin1_b'], act='relu')

    Wl = p['gate_lin_l_W']
    WlxT, Wle = Wl[:, :H].T, Wl[:, H:]
    t = _mm(x0, WlxT, jnp.zeros((H,), _f32))
    s = _rowdots(t, p['gate_att_l'].reshape(1, 128))[0]
    vr = p['gate_lin_r_W'].T @ p['gate_att_r']
    r = _rowdots(x0, vr.reshape(1, 128))[0]

    wq = Wle.T @ p['gate_att_l']
    Mmat = jnp.zeros((128, 256), _f32)
    for j in range(8):
        Mmat = Mmat.at[16 * j:16 * j + 16, 32 * j:32 * j + 16].set(
            jnp.eye(16, dtype=_f32))
        Mmat = Mmat.at[16 * j:16 * j + 16, 32 * j + 17].set(wq)
    onesrow = jnp.zeros((1, 256), _f32)
    for j in range(8):
        onesrow = onesrow.at[0, 32 * j + 16].set(1.0)
    eaq = _build_eaq(edge_attr.reshape(E // 8, 128), Mmat, onesrow)
    eaq = eaq.reshape(E, 32)

    Pg2, QDg = _sc_aggregate(s, r, src, dst, t, N, eaq=eaq)
    x1 = _gate_update(Pg2[0], Pg2[1], QDg[0], QDg[1], x0[:N],
                      Wle.T.reshape(16, 128), p['gate_bias'],
                      p['agru0_Wih'].T, p['agru0_Whh'].T,
                      p['agru0_bih'], p['agru0_bhh'])
    x1 = jnp.pad(x1, ((0, NP - N), (0, 0)))

    hh = _mm(x1, p['aconv1_W'].T, jnp.zeros((H,), _f32))
    sr2 = _rowdots(hh, jnp.stack([p['aconv1_att_src'], p['aconv1_att_dst']]))
    s2, r2 = sr2[0], sr2[1]
    Pa2, Da = _sc_aggregate(s2, r2, src, dst, hh, N)
    x2 = _agg_update(Pa2[0], Pa2[1], Da[0], Da[1], x1[:N],
                     p['aconv1_bias'],
                     p['agru1_Wih'].T, p['agru1_Whh'].T,
                     p['agru1_bih'], p['agru1_bhh'])
    x2 = jnp.pad(x2, ((0, NP - N), (0, 0)))

    BP = 512
    batch_p = jnp.pad(batch, (0, NP - N), constant_values=B).astype(jnp.int32)
    rowids = jnp.arange(NP, dtype=jnp.int32)
    xs = [x0, x1, x2]
    zeroN = jnp.zeros((NP,), _f32)
    P0m2, _ = _sc_aggregate(zeroN, zeroN, rowids, batch_p, x2, 528)
    out = _combine(P0m2[0, :BP], P0m2[1, :BP])
    for i in range(3):
        W = p['mconv%d_W' % i]
        hs = _mm(xs[i], W.T, jnp.zeros((H,), _f32))
        sm = _rowdots(hs, p['mconv%d_att_src' % i].reshape(1, 128))[0]
        hd = _mm(out, W.T, jnp.zeros((H,), _f32), rb=512)
        rm = _rowdots(hd, p['mconv%d_att_dst' % i].reshape(1, 128))[0]
        rm_p = jnp.pad(rm, (0, NP - BP))
        Pm2, Dm = _sc_aggregate(sm, rm_p, rowids, batch_p, hs, 528)
        out = _agg_update(Pm2[0, :BP], Pm2[1, :BP], Dm[0, :BP], Dm[1, :BP], out,
                          p['mconv%d_bias' % i],
                          p['mgru%d_Wih' % i].T, p['mgru%d_Whh' % i].T,
                          p['mgru%d_bih' % i], p['mgru%d_bhh' % i], rb=512)

    return _mm(out, p['lin2_W'].T, p['lin2_b'], rb=512)
